# Initial kernel scaffold; baseline (speedup 1.0000x reference)
import jax
import jax.numpy as jnp
from jax.experimental import pallas as pl


def kernel(x, embedding):
    raise NotImplementedError("write your pallas kernel here")


# speedup vs baseline: 4.9350x; 4.9350x over previous
import functools

import jax
import jax.numpy as jnp
from jax import lax
from jax.experimental import pallas as pl
from jax.experimental.pallas import tpu as pltpu
from jax.experimental.pallas import tpu_sc as plsc

DIM = 64
NC = 2
NS = 16
NW = NC * NS
CHUNK = 256


def _sc_gather(x_flat, table128):
    B = x_flat.shape[0]
    b_per_w = B // NW
    n_chunks = b_per_w // CHUNK
    mesh = plsc.VectorSubcoreMesh(core_axis_name="c", subcore_axis_name="s")

    @functools.partial(
        pl.kernel,
        mesh=mesh,
        out_type=jax.ShapeDtypeStruct((B, DIM), jnp.float32),
        scratch_types=[
            pltpu.VMEM((CHUNK,), jnp.int32),
            pltpu.VMEM((CHUNK, 128), jnp.float32),
            pltpu.VMEM((CHUNK, DIM), jnp.float32),
            pltpu.SemaphoreType.DMA,
        ],
    )
    def k(table_hbm, idx_hbm, out_hbm, idx_v, rows_v, out_v, sem):
        wid = lax.axis_index("s") * NC + lax.axis_index("c")
        base = wid * b_per_w

        def body(g, carry):
            off = base + g * CHUNK
            pltpu.sync_copy(idx_hbm.at[pl.ds(off, CHUNK)], idx_v)
            pltpu.async_copy(table_hbm.at[idx_v], rows_v, sem).wait()

            def compact(r, c):
                for j in range(DIM // 16):
                    out_v[r, pl.ds(j * 16, 16)] = rows_v[r, pl.ds(j * 16, 16)]
                return c

            lax.fori_loop(0, CHUNK, compact, 0)
            pltpu.sync_copy(out_v, out_hbm.at[pl.ds(off, CHUNK)])
            return carry

        lax.fori_loop(0, n_chunks, body, 0)

    return k(table128, x_flat)


def kernel(x, embedding):
    b, h = x.shape
    table128 = jnp.pad(embedding, ((0, 0), (0, 128 - DIM)))
    out = _sc_gather(x.reshape(-1), table128)
    return out.reshape(b, h, DIM)


# speedup vs baseline: 6.2270x; 1.2618x over previous
import functools

import jax
import jax.numpy as jnp
from jax import lax
from jax.experimental import pallas as pl
from jax.experimental.pallas import tpu as pltpu
from jax.experimental.pallas import tpu_sc as plsc

DIM = 64
NC = 2
NS = 16
NW = NC * NS
CHUNK = 128


def _sc_gather(x_flat, table128):
    B = x_flat.shape[0]
    b_per_w = B // NW
    n_chunks = b_per_w // CHUNK
    assert n_chunks % 2 == 0
    mesh = plsc.VectorSubcoreMesh(core_axis_name="c", subcore_axis_name="s")

    @functools.partial(
        pl.kernel,
        mesh=mesh,
        out_type=jax.ShapeDtypeStruct((B, DIM), jnp.float32),
        scratch_types=[
            pltpu.VMEM((2, CHUNK), jnp.int32),
            pltpu.VMEM((2, CHUNK, 128), jnp.float32),
            pltpu.VMEM((2, CHUNK, DIM), jnp.float32),
            pltpu.SemaphoreType.DMA((2,)),
            pltpu.SemaphoreType.DMA((2,)),
            pltpu.SemaphoreType.DMA((2,)),
        ],
    )
    def k(table_hbm, idx_hbm, out_hbm, idx_v, rows_v, out_v,
          sem_i, sem_g, sem_w):
        wid = lax.axis_index("s") * NC + lax.axis_index("c")
        base = wid * b_per_w

        def start_idx(g, b):
            pltpu.async_copy(idx_hbm.at[pl.ds(base + g * CHUNK, CHUNK)],
                             idx_v.at[b], sem_i.at[b])

        def wait_idx(b):
            pltpu.make_async_copy(idx_hbm.at[pl.ds(0, CHUNK)],
                                  idx_v.at[b], sem_i.at[b]).wait()

        def start_gather(b):
            pltpu.async_copy(table_hbm.at[idx_v.at[b]], rows_v.at[b],
                             sem_g.at[b])

        def wait_gather(b):
            pltpu.make_async_copy(table_hbm.at[pl.ds(0, CHUNK)],
                                  rows_v.at[b], sem_g.at[b]).wait()

        def start_write(g, b):
            pltpu.async_copy(out_v.at[b],
                             out_hbm.at[pl.ds(base + g * CHUNK, CHUNK)],
                             sem_w.at[b])

        def wait_write(b):
            pltpu.make_async_copy(out_hbm.at[pl.ds(0, CHUNK)],
                                  out_v.at[b], sem_w.at[b]).wait()

        def compact(b):
            def row(r, c):
                for j in range(DIM // 16):
                    out_v[b, r, pl.ds(j * 16, 16)] = \
                        rows_v[b, r, pl.ds(j * 16, 16)]
                return c

            lax.fori_loop(0, CHUNK, row, 0)

        start_idx(0, 0)
        wait_idx(0)
        start_gather(0)
        start_idx(1, 1)

        def step(g, b):
            wait_gather(b)

            @pl.when(g + 2 < n_chunks)
            def _():
                start_idx(g + 2, b)

            @pl.when(g + 1 < n_chunks)
            def _():
                wait_idx(1 - b)
                start_gather(1 - b)

            @pl.when(g >= 2)
            def _():
                wait_write(b)

            compact(b)
            start_write(g, b)

        def pair(p, c):
            step(2 * p, 0)
            step(2 * p + 1, 1)
            return c

        lax.fori_loop(0, n_chunks // 2, pair, 0)
        wait_write(0)
        wait_write(1)

    return k(table128, x_flat)


def kernel(x, embedding):
    b, h = x.shape
    table128 = jnp.pad(embedding, ((0, 0), (0, 128 - DIM)))
    out = _sc_gather(x.reshape(-1), table128)
    return out.reshape(b, h, DIM)
